# baseline (speedup 1.0000x reference)
import functools

import jax
import jax.numpy as jnp
from jax import lax
from jax.experimental import pallas as pl
from jax.experimental.pallas import tpu as pltpu
from jax.experimental.pallas import tpu_sc as plsc

_B = 16384
_D = 128
_BS = 6144
_BT = _B - _BS
_NC, _NS = 2, 16
_NW = _NC * _NS
_RPW = _BS // _NW
_UROWS = 64
_UNITS = _RPW // _UROWS

_TROWS = 2048
_TC = _TROWS // 128

_AROWS = 2048


def _tc_body(uf_ref, d1f_ref, d2f_ref, v1_ref, v2_ref, o_ref):
    uf = uf_ref[...]
    m1 = jnp.max(uf)
    m2 = jnp.max(uf - d2f_ref[...])
    i = pl.program_id(0)
    d1b = d1f_ref[pl.ds(i * _TC, _TC), :]
    d2b = d2f_ref[pl.ds(i * _TC, _TC), :]
    s1t = jnp.transpose(jnp.minimum(d1b, m2))
    s2t = jnp.transpose(jnp.minimum(d2b, m1))
    for k in range(_TC):
        sl = slice(128 * k, 128 * (k + 1))
        o_ref[sl, :] = (
            v1_ref[sl, :] * s1t[:, k : k + 1] + v2_ref[sl, :] * s2t[:, k : k + 1]
        )


def _sc_body(
    v1_hbm,
    v2_hbm,
    u_hbm,
    d1_hbm,
    d2_hbm,
    o_hbm,
    us,
    ds,
    sd1,
    sd2,
    v1a,
    v2a,
    oa,
    v1b,
    v2b,
    ob,
    s_us,
    s_ds,
    s_sd1,
    s_sd2,
    s_v1a,
    s_v2a,
    s_oa,
    s_v1b,
    s_v2b,
    s_ob,
):
    w = lax.axis_index("s") * _NC + lax.axis_index("c")
    rbase = _BT + w * _RPW
    obase = w * _RPW * _D

    cu = pltpu.async_copy(u_hbm, us, s_us)
    cd = pltpu.async_copy(d2_hbm, ds, s_ds)
    c1 = pltpu.async_copy(d1_hbm.at[pl.ds(rbase, _RPW)], sd1, s_sd1)
    c2 = pltpu.async_copy(d2_hbm.at[pl.ds(rbase, _RPW)], sd2, s_sd2)
    bufs = [(v1a, v2a, oa, s_v1a, s_v2a, s_oa), (v1b, v2b, ob, s_v1b, s_v2b, s_ob)]

    def start_in(t):
        fb = (rbase + t * _UROWS) * _D
        b = bufs[t % 2]
        return (
            pltpu.async_copy(v1_hbm.at[pl.ds(fb, _UROWS * _D)], b[0], b[3]),
            pltpu.async_copy(v2_hbm.at[pl.ds(fb, _UROWS * _D)], b[1], b[4]),
        )

    pend = start_in(0)
    cu.wait()
    cd.wait()

    neg = jnp.full((16,), -3.0e38, jnp.float32)

    def mxstep(k, carry):
        a1, a2 = carry
        base = k * 256
        for j in range(16):
            uu = us[pl.ds(base + 16 * j, 16)]
            dd = ds[pl.ds(base + 16 * j, 16)]
            a1 = jnp.maximum(a1, uu)
            a2 = jnp.maximum(a2, uu - dd)
        return a1, a2

    a1, a2 = lax.fori_loop(0, _B // 256, mxstep, (neg, neg))
    m1 = jnp.max(a1)
    m2 = jnp.max(a2)
    c1.wait()
    c2.wait()
    m1v = jnp.full((16,), m1, jnp.float32)
    m2v = jnp.full((16,), m2, jnp.float32)

    lane = lax.iota(jnp.int32, 16)
    out_pend = [None, None]
    nxt = pend
    for t in range(_UNITS):
        cur = nxt
        b = bufs[t % 2]
        if t + 1 < _UNITS:
            nxt = start_in(t + 1)
        cur[0].wait()
        cur[1].wait()
        if out_pend[t % 2] is not None:
            out_pend[t % 2].wait()

        def group(g, _):
            s1 = jnp.minimum(sd1[pl.ds(t * _UROWS + g * 16, 16)], m2v)
            s2 = jnp.minimum(sd2[pl.ds(t * _UROWS + g * 16, 16)], m1v)
            idx0 = (g * 16 + lane) * _D

            def cols(c8, idx):
                for cc in range(8):
                    x1 = plsc.load_gather(b[0], [idx])
                    x2 = plsc.load_gather(b[1], [idx])
                    plsc.store_scatter(b[2], [idx], x1 * s1 + x2 * s2)
                    idx = idx + 1
                return idx

            lax.fori_loop(0, _D // 8, cols, idx0)
            return 0

        lax.fori_loop(0, _UROWS // 16, group, 0)
        out_pend[t % 2] = pltpu.async_copy(
            b[2], o_hbm.at[pl.ds(obase + t * _UROWS * _D, _UROWS * _D)], b[5]
        )
    for op_ in out_pend:
        if op_ is not None:
            op_.wait()


def _asm_body(sc_ref, tc_ref, o_ref):
    del tc_ref
    o_ref[...] = sc_ref[...]


def kernel(u, d1, d2, v1, v2):
    B, D = v1.shape
    uf = u.reshape(B // 128, 128)
    d1f = d1.reshape(B // 128, 128)
    d2f = d2.reshape(B // 128, 128)

    tc_full = pl.pallas_call(
        _tc_body,
        grid=(_BT // _TROWS,),
        in_specs=[
            pl.BlockSpec((B // 128, 128), lambda i: (0, 0)),
            pl.BlockSpec((B // 128, 128), lambda i: (0, 0)),
            pl.BlockSpec((B // 128, 128), lambda i: (0, 0)),
            pl.BlockSpec((_TROWS, D), lambda i: (i, 0)),
            pl.BlockSpec((_TROWS, D), lambda i: (i, 0)),
        ],
        out_specs=pl.BlockSpec((_TROWS, D), lambda i: (i, 0)),
        out_shape=jax.ShapeDtypeStruct((B, D), jnp.float32),
    )(uf, d1f, d2f, v1, v2)

    sc_kernel = functools.partial(
        pl.kernel,
        mesh=plsc.VectorSubcoreMesh(core_axis_name="c", subcore_axis_name="s"),
        compiler_params=pltpu.CompilerParams(needs_layout_passes=False),
        out_type=jax.ShapeDtypeStruct((_BS * D,), jnp.float32),
        scratch_types=[
            pltpu.VMEM((B,), jnp.float32),
            pltpu.VMEM((B,), jnp.float32),
            pltpu.VMEM((_RPW,), jnp.float32),
            pltpu.VMEM((_RPW,), jnp.float32),
            pltpu.VMEM((_UROWS * _D,), jnp.float32),
            pltpu.VMEM((_UROWS * _D,), jnp.float32),
            pltpu.VMEM((_UROWS * _D,), jnp.float32),
            pltpu.VMEM((_UROWS * _D,), jnp.float32),
            pltpu.VMEM((_UROWS * _D,), jnp.float32),
            pltpu.VMEM((_UROWS * _D,), jnp.float32),
            pltpu.SemaphoreType.DMA,
            pltpu.SemaphoreType.DMA,
            pltpu.SemaphoreType.DMA,
            pltpu.SemaphoreType.DMA,
            pltpu.SemaphoreType.DMA,
            pltpu.SemaphoreType.DMA,
            pltpu.SemaphoreType.DMA,
            pltpu.SemaphoreType.DMA,
            pltpu.SemaphoreType.DMA,
            pltpu.SemaphoreType.DMA,
        ],
    )
    sc_out = sc_kernel(_sc_body)(
        v1.reshape(B * D),
        v2.reshape(B * D),
        u.reshape(B),
        d1.reshape(B),
        d2.reshape(B),
    )

    out = pl.pallas_call(
        _asm_body,
        grid=(_BS // _AROWS,),
        in_specs=[
            pl.BlockSpec((_AROWS, D), lambda i: (i, 0)),
            pl.BlockSpec(memory_space=pl.ANY),
        ],
        out_specs=pl.BlockSpec((_AROWS, D), lambda i: (_BT // _AROWS + i, 0)),
        out_shape=jax.ShapeDtypeStruct((B, D), jnp.float32),
        input_output_aliases={1: 0},
    )(sc_out.reshape(_BS, D), tc_full)
    return out


# speedup vs baseline: 2.1300x; 2.1300x over previous
import functools

import jax
import jax.numpy as jnp
from jax import lax
from jax.experimental import pallas as pl
from jax.experimental.pallas import tpu as pltpu
from jax.experimental.pallas import tpu_sc as plsc

_B = 16384
_D = 128
_BS = 6144
_BT = _B - _BS
_NC, _NS = 2, 16
_NW = _NC * _NS
_RPW = _BS // _NW
_UROWS = 64
_UNITS = _RPW // _UROWS

_TROWS = 2048
_TC = _TROWS // 128

_AROWS = 2048


def _tc_body(uf_ref, d1f_ref, d2f_ref, v1_ref, v2_ref, o_ref):
    uf = uf_ref[...]
    m1 = jnp.max(uf)
    m2 = jnp.max(uf - d2f_ref[...])
    i = pl.program_id(0)
    d1b = d1f_ref[pl.ds(i * _TC, _TC), :]
    d2b = d2f_ref[pl.ds(i * _TC, _TC), :]
    s1t = jnp.transpose(jnp.minimum(d1b, m2))
    s2t = jnp.transpose(jnp.minimum(d2b, m1))
    for k in range(_TC):
        sl = slice(128 * k, 128 * (k + 1))
        o_ref[sl, :] = (
            v1_ref[sl, :] * s1t[:, k : k + 1] + v2_ref[sl, :] * s2t[:, k : k + 1]
        )


def _sc_body(
    v1_hbm,
    v2_hbm,
    u_hbm,
    d1_hbm,
    d2_hbm,
    o_hbm,
    us,
    ds,
    sd1,
    sd2,
    v1a,
    v2a,
    oa,
    v1b,
    v2b,
    ob,
    sb1,
    sb2,
    s_us,
    s_ds,
    s_sd1,
    s_sd2,
    s_v1a,
    s_v2a,
    s_oa,
    s_v1b,
    s_v2b,
    s_ob,
):
    w = lax.axis_index("s") * _NC + lax.axis_index("c")
    rbase = _BT + w * _RPW
    obase = w * _RPW * _D

    cu = pltpu.async_copy(u_hbm, us, s_us)
    cd = pltpu.async_copy(d2_hbm, ds, s_ds)
    c1 = pltpu.async_copy(d1_hbm.at[pl.ds(rbase, _RPW)], sd1, s_sd1)
    c2 = pltpu.async_copy(d2_hbm.at[pl.ds(rbase, _RPW)], sd2, s_sd2)
    bufs = [(v1a, v2a, oa, s_v1a, s_v2a, s_oa), (v1b, v2b, ob, s_v1b, s_v2b, s_ob)]

    def start_in(t):
        fb = (rbase + t * _UROWS) * _D
        b = bufs[t % 2]
        return (
            pltpu.async_copy(v1_hbm.at[pl.ds(fb, _UROWS * _D)], b[0], b[3]),
            pltpu.async_copy(v2_hbm.at[pl.ds(fb, _UROWS * _D)], b[1], b[4]),
        )

    pend = start_in(0)
    cu.wait()
    cd.wait()

    neg = jnp.full((16,), -3.0e38, jnp.float32)

    def mxstep(k, carry):
        a1, a2 = carry
        base = k * 256
        for j in range(16):
            uu = us[pl.ds(base + 16 * j, 16)]
            dd = ds[pl.ds(base + 16 * j, 16)]
            a1 = jnp.maximum(a1, uu)
            a2 = jnp.maximum(a2, uu - dd)
        return a1, a2

    a1, a2 = lax.fori_loop(0, _B // 256, mxstep, (neg, neg))
    m1 = jnp.max(a1)
    m2 = jnp.max(a2)
    c1.wait()
    c2.wait()
    m1v = jnp.full((16,), m1, jnp.float32)
    m2v = jnp.full((16,), m2, jnp.float32)

    def expand(g, _):
        s1 = jnp.minimum(sd1[pl.ds(g * 16, 16)], m2v)
        s2 = jnp.minimum(sd2[pl.ds(g * 16, 16)], m1v)
        for t16 in range(16):
            sb1[pl.ds((g * 16 + t16) * 16, 16)] = jnp.full((16,), s1[t16], jnp.float32)
            sb2[pl.ds((g * 16 + t16) * 16, 16)] = jnp.full((16,), s2[t16], jnp.float32)
        return 0

    lax.fori_loop(0, _RPW // 16, expand, 0)

    out_pend = [None, None]
    nxt = pend
    for t in range(_UNITS):
        cur = nxt
        b = bufs[t % 2]
        if t + 1 < _UNITS:
            nxt = start_in(t + 1)
        cur[0].wait()
        cur[1].wait()
        if out_pend[t % 2] is not None:
            out_pend[t % 2].wait()

        def row(r, _):
            a = sb1[pl.ds((t * _UROWS + r) * 16, 16)]
            c = sb2[pl.ds((t * _UROWS + r) * 16, 16)]
            for j in range(8):
                cs = pl.ds(r * _D + 16 * j, 16)
                b[2][cs] = b[0][cs] * a + b[1][cs] * c
            return 0

        lax.fori_loop(0, _UROWS, row, 0)
        out_pend[t % 2] = pltpu.async_copy(
            b[2], o_hbm.at[pl.ds(obase + t * _UROWS * _D, _UROWS * _D)], b[5]
        )
    for op_ in out_pend:
        if op_ is not None:
            op_.wait()


def _asm_body(sc_ref, tc_ref, o_ref):
    del tc_ref
    o_ref[...] = sc_ref[...]


def kernel(u, d1, d2, v1, v2):
    B, D = v1.shape
    uf = u.reshape(B // 128, 128)
    d1f = d1.reshape(B // 128, 128)
    d2f = d2.reshape(B // 128, 128)

    tc_full = pl.pallas_call(
        _tc_body,
        grid=(_BT // _TROWS,),
        in_specs=[
            pl.BlockSpec((B // 128, 128), lambda i: (0, 0)),
            pl.BlockSpec((B // 128, 128), lambda i: (0, 0)),
            pl.BlockSpec((B // 128, 128), lambda i: (0, 0)),
            pl.BlockSpec((_TROWS, D), lambda i: (i, 0)),
            pl.BlockSpec((_TROWS, D), lambda i: (i, 0)),
        ],
        out_specs=pl.BlockSpec((_TROWS, D), lambda i: (i, 0)),
        out_shape=jax.ShapeDtypeStruct((B, D), jnp.float32),
    )(uf, d1f, d2f, v1, v2)

    sc_kernel = functools.partial(
        pl.kernel,
        mesh=plsc.VectorSubcoreMesh(core_axis_name="c", subcore_axis_name="s"),
        compiler_params=pltpu.CompilerParams(needs_layout_passes=False),
        out_type=jax.ShapeDtypeStruct((_BS * D,), jnp.float32),
        scratch_types=[
            pltpu.VMEM((B,), jnp.float32),
            pltpu.VMEM((B,), jnp.float32),
            pltpu.VMEM((_RPW,), jnp.float32),
            pltpu.VMEM((_RPW,), jnp.float32),
            pltpu.VMEM((_UROWS * _D,), jnp.float32),
            pltpu.VMEM((_UROWS * _D,), jnp.float32),
            pltpu.VMEM((_UROWS * _D,), jnp.float32),
            pltpu.VMEM((_UROWS * _D,), jnp.float32),
            pltpu.VMEM((_UROWS * _D,), jnp.float32),
            pltpu.VMEM((_UROWS * _D,), jnp.float32),
            pltpu.VMEM((_RPW * 16,), jnp.float32),
            pltpu.VMEM((_RPW * 16,), jnp.float32),
            pltpu.SemaphoreType.DMA,
            pltpu.SemaphoreType.DMA,
            pltpu.SemaphoreType.DMA,
            pltpu.SemaphoreType.DMA,
            pltpu.SemaphoreType.DMA,
            pltpu.SemaphoreType.DMA,
            pltpu.SemaphoreType.DMA,
            pltpu.SemaphoreType.DMA,
            pltpu.SemaphoreType.DMA,
            pltpu.SemaphoreType.DMA,
        ],
    )
    sc_out = sc_kernel(_sc_body)(
        v1.reshape(B * D),
        v2.reshape(B * D),
        u.reshape(B),
        d1.reshape(B),
        d2.reshape(B),
    )

    out = pl.pallas_call(
        _asm_body,
        grid=(_BS // _AROWS,),
        in_specs=[
            pl.BlockSpec((_AROWS, D), lambda i: (i, 0)),
            pl.BlockSpec(memory_space=pl.ANY),
        ],
        out_specs=pl.BlockSpec((_AROWS, D), lambda i: (_BT // _AROWS + i, 0)),
        out_shape=jax.ShapeDtypeStruct((B, D), jnp.float32),
        input_output_aliases={1: 0},
    )(sc_out.reshape(_BS, D), tc_full)
    return out


# speedup vs baseline: 6.7064x; 3.1485x over previous
import jax
import jax.numpy as jnp
from jax.experimental import pallas as pl

_ROWS = 8192
_C = _ROWS // 128


def _body(uf_ref, d1f_ref, d2f_ref, v1_ref, v2_ref, o_ref):
    uf = uf_ref[...]
    m1 = jnp.max(uf)
    m2 = jnp.max(uf - d2f_ref[...])
    i = pl.program_id(0)
    d1b = d1f_ref[pl.ds(i * _C, _C), :]
    d2b = d2f_ref[pl.ds(i * _C, _C), :]
    s1t = jnp.transpose(jnp.minimum(d1b, m2))
    s2t = jnp.transpose(jnp.minimum(d2b, m1))
    ones_row = jnp.ones((1, 128), jnp.float32)
    for k in range(_C):
        sl = slice(128 * k, 128 * (k + 1))
        s1b = jax.lax.dot(s1t[:, k : k + 1], ones_row)
        s2b = jax.lax.dot(s2t[:, k : k + 1], ones_row)
        o_ref[sl, :] = v1_ref[sl, :] * s1b + v2_ref[sl, :] * s2b


def kernel(u, d1, d2, v1, v2):
    B, D = v1.shape
    uf = u.reshape(B // 128, 128)
    d1f = d1.reshape(B // 128, 128)
    d2f = d2.reshape(B // 128, 128)
    grid = (B // _ROWS,)
    out = pl.pallas_call(
        _body,
        grid=grid,
        in_specs=[
            pl.BlockSpec((B // 128, 128), lambda i: (0, 0)),
            pl.BlockSpec((B // 128, 128), lambda i: (0, 0)),
            pl.BlockSpec((B // 128, 128), lambda i: (0, 0)),
            pl.BlockSpec((_ROWS, D), lambda i: (i, 0)),
            pl.BlockSpec((_ROWS, D), lambda i: (i, 0)),
        ],
        out_specs=pl.BlockSpec((_ROWS, D), lambda i: (i, 0)),
        out_shape=jax.ShapeDtypeStruct((B, D), jnp.float32),
    )(uf, d1f, d2f, v1, v2)
    return out
